# baseline (speedup 1.0000x reference)
import functools

import jax
import jax.numpy as jnp
from jax import lax
from jax.experimental import pallas as pl
from jax.experimental.pallas import tpu as pltpu
from jax.experimental.pallas import tpu_sc as plsc

_BATCH = 16384
_RANK = 64
_LANES = 16

_info = plsc.get_sparse_core_info()
_NC, _NS = _info.num_cores, _info.num_subcores
_NW = _NC * _NS
_CHUNK = _BATCH // _NW


def _distmult_body(s_idx_hbm, p_idx_hbm, o_idx_hbm, emb_so_hbm, emb_p_hbm,
                   out_hbm, s_idx_v, p_idx_v, o_idx_v, s_rows, p_rows,
                   o_rows, part_v, out_v, sem):
    wid = lax.axis_index("s") * _NC + lax.axis_index("c")
    base = wid * _CHUNK

    pltpu.sync_copy(s_idx_hbm.at[pl.ds(base, _CHUNK)], s_idx_v)
    pltpu.sync_copy(p_idx_hbm.at[pl.ds(base, _CHUNK)], p_idx_v)
    pltpu.sync_copy(o_idx_hbm.at[pl.ds(base, _CHUNK)], o_idx_v)

    cp_s = pltpu.make_async_copy(emb_so_hbm.at[s_idx_v], s_rows, sem)
    cp_p = pltpu.make_async_copy(emb_p_hbm.at[p_idx_v], p_rows, sem)
    cp_o = pltpu.make_async_copy(emb_so_hbm.at[o_idx_v], o_rows, sem)
    cp_s.start()
    cp_p.start()
    cp_o.start()
    cp_s.wait()
    cp_p.wait()
    cp_o.wait()

    lane = lax.iota(jnp.int32, _LANES)

    def body(g, _):
        base_b = g * _LANES
        for j in range(_LANES):
            b = base_b + j
            acc = (s_rows[b, pl.ds(0, _LANES)]
                   * p_rows[b, pl.ds(0, _LANES)]
                   * o_rows[b, pl.ds(0, _LANES)])
            for k in range(1, _RANK // _LANES):
                acc = acc + (s_rows[b, pl.ds(k * _LANES, _LANES)]
                             * p_rows[b, pl.ds(k * _LANES, _LANES)]
                             * o_rows[b, pl.ds(k * _LANES, _LANES)])
            part_v[j, pl.ds(0, _LANES)] = acc
        out_vec = plsc.load_gather(part_v, [lane, jnp.full((_LANES,), 0,
                                                           jnp.int32)])
        for i in range(1, _LANES):
            out_vec = out_vec + plsc.load_gather(
                part_v, [lane, jnp.full((_LANES,), i, jnp.int32)])
        out_v[pl.ds(base_b, _LANES)] = out_vec
        return _

    lax.fori_loop(0, _CHUNK // _LANES, body, None)

    pltpu.sync_copy(out_v, out_hbm.at[pl.ds(base, _CHUNK)])


@jax.jit
def kernel(s_idx, p_idx, o_idx, emb_so, emb_p):
    mesh = plsc.VectorSubcoreMesh(core_axis_name="c", subcore_axis_name="s")
    run = pl.kernel(
        _distmult_body,
        out_type=jax.ShapeDtypeStruct((_BATCH,), jnp.float32),
        mesh=mesh,
        compiler_params=pltpu.CompilerParams(needs_layout_passes=False,
                                             use_tc_tiling_on_sc=False),
        scratch_types=[
            pltpu.VMEM((_CHUNK,), jnp.int32),
            pltpu.VMEM((_CHUNK,), jnp.int32),
            pltpu.VMEM((_CHUNK,), jnp.int32),
            pltpu.VMEM((_CHUNK, _RANK), jnp.float32),
            pltpu.VMEM((_CHUNK, _RANK), jnp.float32),
            pltpu.VMEM((_CHUNK, _RANK), jnp.float32),
            pltpu.VMEM((_LANES, _LANES), jnp.float32),
            pltpu.VMEM((_CHUNK,), jnp.float32),
            pltpu.SemaphoreType.DMA,
        ],
    )
    return run(s_idx.astype(jnp.int32), p_idx.astype(jnp.int32),
               o_idx.astype(jnp.int32), emb_so, emb_p)


# speedup vs baseline: 1.6644x; 1.6644x over previous
import functools

import jax
import jax.numpy as jnp
from jax import lax
from jax.experimental import pallas as pl
from jax.experimental.pallas import tpu as pltpu
from jax.experimental.pallas import tpu_sc as plsc

_BATCH = 16384
_RANK = 64
_LANES = 16
_BLK = 16

_info = plsc.get_sparse_core_info()
_NC, _NS = _info.num_cores, _info.num_subcores
_NW = _NC * _NS
_CHUNK = _BATCH // _NW
_NBLK = _CHUNK // _BLK


def _distmult_body(s_idx_hbm, p_idx_hbm, o_idx_hbm, emb_so_hbm, emb_p_hbm,
                   out_hbm, s_idx_sm, p_idx_sm, o_idx_sm, idx_stage, s_rows,
                   p_rows, o_rows, part_v, out_v, sem0, sem1):
    wid = lax.axis_index("s") * _NC + lax.axis_index("c")
    base = wid * _CHUNK

    pltpu.sync_copy(s_idx_hbm.at[pl.ds(base, _CHUNK)], s_idx_sm)
    pltpu.sync_copy(p_idx_hbm.at[pl.ds(base, _CHUNK)], p_idx_sm)
    pltpu.sync_copy(o_idx_hbm.at[pl.ds(base, _CHUNK)], o_idx_sm)

    sems = [sem0, sem1]
    lane = lax.iota(jnp.int32, _LANES)

    def fire(g, slot, sem):
        b0 = g * _BLK
        ev_s = s_idx_sm[pl.ds(b0, _LANES)]
        ev_p = p_idx_sm[pl.ds(b0, _LANES)]
        ev_o = o_idx_sm[pl.ds(b0, _LANES)]
        for j in range(_BLK):
            pltpu.make_async_copy(
                emb_so_hbm.at[pl.ds(ev_s[j], 1)],
                s_rows.at[slot, pl.ds(j, 1)], sem).start()
            pltpu.make_async_copy(
                emb_p_hbm.at[pl.ds(ev_p[j], 1)],
                p_rows.at[slot, pl.ds(j, 1)], sem).start()
            pltpu.make_async_copy(
                emb_so_hbm.at[pl.ds(ev_o[j], 1)],
                o_rows.at[slot, pl.ds(j, 1)], sem).start()

    def drain(slot, sem):
        for j in range(_BLK):
            pltpu.make_async_copy(
                emb_so_hbm.at[pl.ds(0, 1)],
                s_rows.at[slot, pl.ds(j, 1)], sem).wait()
            pltpu.make_async_copy(
                emb_p_hbm.at[pl.ds(0, 1)],
                p_rows.at[slot, pl.ds(j, 1)], sem).wait()
            pltpu.make_async_copy(
                emb_so_hbm.at[pl.ds(0, 1)],
                o_rows.at[slot, pl.ds(j, 1)], sem).wait()

    def compute(g, slot):
        for j in range(_BLK):
            acc = (s_rows[slot, j, pl.ds(0, _LANES)]
                   * p_rows[slot, j, pl.ds(0, _LANES)]
                   * o_rows[slot, j, pl.ds(0, _LANES)])
            for k in range(1, _RANK // _LANES):
                acc = acc + (s_rows[slot, j, pl.ds(k * _LANES, _LANES)]
                             * p_rows[slot, j, pl.ds(k * _LANES, _LANES)]
                             * o_rows[slot, j, pl.ds(k * _LANES, _LANES)])
            part_v[j, pl.ds(0, _LANES)] = acc
        out_vec = plsc.load_gather(part_v, [lane, jnp.full((_LANES,), 0,
                                                           jnp.int32)])
        for i in range(1, _LANES):
            out_vec = out_vec + plsc.load_gather(
                part_v, [lane, jnp.full((_LANES,), i, jnp.int32)])
        out_v[pl.ds(g * _BLK, _LANES)] = out_vec

    def body(g, _):
        slot = lax.rem(g, 2)

        @pl.when(g < _NBLK)
        def _fire():
            @pl.when(slot == 0)
            def _():
                fire(g, 0, sems[0])
            @pl.when(slot == 1)
            def _():
                fire(g, 1, sems[1])

        @pl.when(g > 0)
        def _consume():
            pslot = lax.rem(g + 1, 2)

            @pl.when(pslot == 0)
            def _():
                drain(0, sems[0])
                compute(g - 1, 0)
            @pl.when(pslot == 1)
            def _():
                drain(1, sems[1])
                compute(g - 1, 1)
        return _

    lax.fori_loop(0, _NBLK + 1, body, None)

    pltpu.sync_copy(out_v, out_hbm.at[pl.ds(base, _CHUNK)])


@jax.jit
def kernel(s_idx, p_idx, o_idx, emb_so, emb_p):
    mesh = plsc.VectorSubcoreMesh(core_axis_name="c", subcore_axis_name="s")
    run = pl.kernel(
        _distmult_body,
        out_type=jax.ShapeDtypeStruct((_BATCH,), jnp.float32),
        mesh=mesh,
        compiler_params=pltpu.CompilerParams(needs_layout_passes=False,
                                             use_tc_tiling_on_sc=True),
        scratch_types=[
            pltpu.VMEM((_CHUNK,), jnp.int32),
            pltpu.VMEM((_CHUNK,), jnp.int32),
            pltpu.VMEM((_CHUNK,), jnp.int32),
            pltpu.VMEM((_CHUNK,), jnp.int32),
            pltpu.VMEM((2, _BLK, _RANK), jnp.float32),
            pltpu.VMEM((2, _BLK, _RANK), jnp.float32),
            pltpu.VMEM((2, _BLK, _RANK), jnp.float32),
            pltpu.VMEM((_LANES, _LANES), jnp.float32),
            pltpu.VMEM((_CHUNK,), jnp.float32),
            pltpu.SemaphoreType.DMA,
            pltpu.SemaphoreType.DMA,
        ],
    )
    return run(s_idx.astype(jnp.int32), p_idx.astype(jnp.int32),
               o_idx.astype(jnp.int32), emb_so, emb_p)
